# baseline (speedup 1.0000x reference)
import jax
import jax.numpy as jnp
from jax.experimental import pallas as pl
from jax.experimental.pallas import tpu as pltpu

_N = 2048
_D = 512
_DOUT = 128
_ALPHA = 0.0005
_R = 1024
_NBLK = _N // _R
_IMAX = 0x7FFFFFFF
_HIGH = -65536


def _kern(x_ref, yo_ref, yb_ref, w_ref, b_ref,
          out_ref, loss_sm, s_sm, xT_vm, yoT_vm, xsq_vm, code_vm):
    i = pl.program_id(0)

    @pl.when(i == 0)
    def _init():
        x = x_ref[...]
        xT_vm[...] = x.astype(jnp.bfloat16).T
        yoT_vm[...] = yo_ref[...].astype(jnp.bfloat16).T
        xTb = xT_vm[...]
        xsq_vm[...] = jnp.sum((xTb * xTb).astype(jnp.float32),
                              axis=0, keepdims=True)
        ybT = yb_ref[...].T
        code_vm[...] = (jax.lax.broadcasted_iota(jnp.int32, (1, _N), 1) * 16
                        + ybT.astype(jnp.int32))
        net = jnp.dot(x, w_ref[...], preferred_element_type=jnp.float32)
        err = net + b_ref[0, 0] - yb_ref[...]
        loss_sm[0, 0] = jnp.sum(err * err) * (1.0 / _N)

        yoT = yoT_vm[...]
        ysq_all = jnp.sum((yoT * yoT).astype(jnp.float32),
                          axis=0, keepdims=True)

        def body(c, acc):
            yo = yo_ref[pl.ds(c * _R, _R), :]
            ysq_c = jnp.sum(yo * yo, axis=1, keepdims=True)
            d2y = ysq_c + ysq_all - 2.0 * jnp.dot(
                yo.astype(jnp.bfloat16), yoT,
                preferred_element_type=jnp.float32)
            return acc + jnp.sum(jnp.sqrt(jnp.maximum(d2y, 0.0) + 1e-12),
                                 axis=0, keepdims=True)

        scol = jax.lax.fori_loop(
            0, _NBLK, body, jnp.zeros((1, _N), jnp.float32))
        s_sm[0, 0] = jnp.sum(scol)

    @pl.when(i > 0)
    def _phase_a():
        blk = i - 1
        x = x_ref[pl.ds(blk * _R, _R), :]
        xsq_blk = jnp.sum(x * x, axis=1, keepdims=True)
        x2b = (x + x).astype(jnp.bfloat16)
        t = xsq_vm[...] - jnp.dot(
            x2b, xT_vm[...], preferred_element_type=jnp.float32)

        col = jax.lax.broadcasted_iota(jnp.int32, (_R, _N), 1)
        rowg = jax.lax.broadcasted_iota(jnp.int32, (_R, _N), 0) + blk * _R
        u = jax.lax.bitcast_convert_type(t, jnp.int32)
        key = (u & _HIGH) | code_vm[...]
        key = jnp.where(col == rowg, _IMAX, key)
        k1 = jnp.min(key, axis=1, keepdims=True)
        c1 = key == k1
        key2 = jnp.where(c1, _IMAX, key)
        k2 = jnp.min(key2, axis=1, keepdims=True)
        c2 = key2 == k2

        ybi = yb_ref[pl.ds(blk * _R, _R), :].astype(jnp.int32)

        def _weight(k):
            lbl = k & 0xF
            v = jax.lax.bitcast_convert_type(k & _HIGH, jnp.float32)
            d = jnp.sqrt(jnp.maximum(v + xsq_blk, 0.0) + 1e-12)
            return jnp.where(lbl == ybi, jnp.exp(-d), 0.0)

        base = loss_sm[0, 0]
        coef = _ALPHA * s_sm[0, 0]
        a1 = base + coef * _weight(k1)
        a2 = base + coef * _weight(k2)
        out_ref[...] = jnp.where(c1, a1, jnp.where(c2, a2, base))


def kernel(x_batch, y_batch, y_output, W, b):
    first = lambda i: (0, 0)
    ablk = lambda i: (jnp.maximum(i - 1, 0), 0)
    out = pl.pallas_call(
        _kern,
        grid=(_NBLK + 1,),
        in_specs=[
            pl.BlockSpec((_N, _D), first),
            pl.BlockSpec((_N, _DOUT), first),
            pl.BlockSpec((_N, 1), first),
            pl.BlockSpec((_D, 1), first),
            pl.BlockSpec((1, 1), first),
        ],
        out_specs=pl.BlockSpec((_R, _N), ablk),
        out_shape=jax.ShapeDtypeStruct((_N, _N), jnp.float32),
        scratch_shapes=[
            pltpu.SMEM((1, 1), jnp.float32),
            pltpu.SMEM((1, 1), jnp.float32),
            pltpu.VMEM((_D, _N), jnp.bfloat16),
            pltpu.VMEM((_DOUT, _N), jnp.bfloat16),
            pltpu.VMEM((1, _N), jnp.float32),
            pltpu.VMEM((1, _N), jnp.int32),
        ],
    )(x_batch, y_output, y_batch, W, b.reshape(1, 1))
    return out
